# baseline (speedup 1.0000x reference)
import functools

import jax
import jax.numpy as jnp
from jax import lax
from jax.experimental import pallas as pl
from jax.experimental.pallas import tpu as pltpu
from jax.experimental.pallas import tpu_sc as plsc

NC = 2
NS = 16
NW = NC * NS

BATCH = 4096
SEQ = 200
D = 64
BPW = BATCH // NW
CA, CB = 128, 72
NBUF = 4


def _mesh():
    return plsc.VectorSubcoreMesh(core_axis_name="c", subcore_axis_name="s")


@functools.partial(
    pl.kernel,
    out_type=jax.ShapeDtypeStruct((BATCH, SEQ, D), jnp.float32),
    mesh=_mesh(),
    scratch_types=[
        pltpu.VMEM((BPW, SEQ), jnp.int32),
        *[pltpu.VMEM((SEQ, D), jnp.float32) for _ in range(NBUF)],
        *[pltpu.SemaphoreType.DMA for _ in range(2 * NBUF)],
    ],
    compiler_params=pltpu.CompilerParams(use_tc_tiling_on_sc=False),
)
def _embed_lookup(x_hbm, table_hbm, out_hbm, idx_v, *bufs_sems):
    rows = bufs_sems[:NBUF]
    gsem = bufs_sems[NBUF : 2 * NBUF]
    osem = bufs_sems[2 * NBUF :]
    wid = lax.axis_index("s") * NC + lax.axis_index("c")
    b0 = wid * BPW

    pltpu.sync_copy(x_hbm.at[pl.ds(b0, BPW)], idx_v)

    def start_gathers(b, slot):
        pltpu.async_copy(
            table_hbm.at[idx_v.at[b, pl.ds(0, CA)]],
            rows[slot].at[pl.ds(0, CA)],
            gsem[slot],
        )
        pltpu.async_copy(
            table_hbm.at[idx_v.at[b, pl.ds(CA, CB)]],
            rows[slot].at[pl.ds(CA, CB)],
            gsem[slot],
        )

    def wait_gathers(b, slot):
        pltpu.make_async_copy(
            table_hbm.at[idx_v.at[b, pl.ds(0, CA)]],
            rows[slot].at[pl.ds(0, CA)],
            gsem[slot],
        ).wait()
        pltpu.make_async_copy(
            table_hbm.at[idx_v.at[b, pl.ds(CA, CB)]],
            rows[slot].at[pl.ds(CA, CB)],
            gsem[slot],
        ).wait()

    for slot in range(NBUF):
        start_gathers(slot, slot)

    @pl.loop(0, BPW, step=NBUF)
    def _(j):
        for slot in range(NBUF):
            b = j + slot
            wait_gathers(b, slot)
            pltpu.async_copy(rows[slot], out_hbm.at[b0 + b], osem[slot])
        for slot in range(NBUF):
            b = j + slot
            pltpu.make_async_copy(
                rows[slot], out_hbm.at[b0 + b], osem[slot]
            ).wait()

            @pl.when(b + NBUF < BPW)
            def _():
                start_gathers(b + NBUF, slot)


def kernel(x, embed_word):
    return _embed_lookup(x, embed_word)
